# baseline (speedup 1.0000x reference)
import functools

import jax
import jax.numpy as jnp
from jax import lax
from jax.experimental import pallas as pl
from jax.experimental.pallas import tpu as pltpu
from jax.experimental.pallas import tpu_sc as plsc

N = 10000
E = 320000
D = 128
NP = 10240
HALF = NP // 2
NTILES = 16
RT = HALF // NTILES
EPT = E // NTILES
SCH = 4000
NSC = EPT // SCH
GC = 128
CAP = 160
EC = 32
TRASH = HALF
SHIFT = 13
LOWM = (1 << SHIFT) - 1
ALPHA = 0.2
AGGR = 0.5

f32 = jnp.float32
i32 = jnp.int32

_mesh = plsc.VectorSubcoreMesh(core_axis_name="c", subcore_axis_name="s")


def _mm_body(x_ref, wt_ref, b_ref, o_ref):
    o_ref[...] = (
        jnp.dot(x_ref[...], wt_ref[...], preferred_element_type=f32)
        + b_ref[0:1, :]
    )


def _mm(xp, Wt, b8):
    BLK = 512
    return pl.pallas_call(
        _mm_body,
        grid=(NP // BLK,),
        in_specs=[
            pl.BlockSpec((BLK, D), lambda i: (i, 0)),
            pl.BlockSpec((D, D), lambda i: (0, 0)),
            pl.BlockSpec((8, D), lambda i: (0, 0)),
        ],
        out_specs=pl.BlockSpec((BLK, D), lambda i: (i, 0)),
        out_shape=jax.ShapeDtypeStruct((NP, D), f32),
    )(xp, Wt, b8)


def _k2_body(h0_hbm, dst_hbm, ds_hbm, g0_hbm,
             dst_s, degl, deg_sh, cmb, ds_t, h_t, g_t):
    cid = lax.axis_index("c")
    sid = lax.axis_index("s")
    lo = cid * HALF
    hi = lo + HALF
    zero16 = jnp.zeros((16,), f32)
    ones16 = jnp.ones((16,), f32)

    def zb(i, c):
        degl[pl.ds(i * 16, 16)] = zero16
        return c
    lax.fori_loop(0, HALF // 16, zb, 0)

    ebase = sid * EPT
    pltpu.sync_copy(dst_hbm.at[pl.ds(ebase, EPT)], dst_s)

    def it(i, c):
        d16 = dst_s[pl.ds(i * 16, 16)]
        m = (d16 >= lo) & (d16 < hi)
        plsc.addupdate_scatter(degl, [d16 - lo], ones16, mask=m)
        return c
    lax.fori_loop(0, EPT // 16, it, 0)

    pltpu.sync_copy(degl, deg_sh.at[pl.ds(sid * HALF, HALF)])
    plsc.subcore_barrier()
    for t in range(NTILES):
        pltpu.sync_copy(deg_sh.at[pl.ds(t * HALF + sid * RT, RT)],
                        cmb.at[pl.ds(t * RT, RT)])
    for k in range(RT // 16):
        acc = cmb[pl.ds(k * 16, 16)]
        for t in range(1, NTILES):
            acc = acc + cmb[pl.ds(t * RT + k * 16, 16)]
        dg = jnp.maximum(acc, 1.0)
        xi = plsc.bitcast(dg, i32)
        yi = 0x5F3759DF - lax.shift_right_arithmetic(xi, 1)
        y = plsc.bitcast(yi, f32)
        for _ in range(3):
            y = y * (1.5 - 0.5 * dg * y * y)
        ds_t[pl.ds(k * 16, 16)] = y
    gbase = lo + sid * RT
    pltpu.sync_copy(ds_t, ds_hbm.at[pl.ds(gbase, RT)])

    for cb in range(RT // 64):
        pltpu.sync_copy(h0_hbm.at[pl.ds(gbase + cb * 64, 64)], h_t)

        def rb(r, c):
            dsr = plsc.load_gather(ds_t, [jnp.full((16,), cb * 64 + r, i32)])
            for v in range(D // 16):
                g_t[r, pl.ds(v * 16, 16)] = h_t[r, pl.ds(v * 16, 16)] * dsr
            return c
        lax.fori_loop(0, 64, rb, 0)
        pltpu.sync_copy(g_t, g0_hbm.at[pl.ds(gbase + cb * 64, 64)])


_k2 = functools.partial(
    pl.kernel,
    out_type=[
        jax.ShapeDtypeStruct((NP,), f32),
        jax.ShapeDtypeStruct((NP, D), f32),
    ],
    mesh=_mesh,
    compiler_params=pltpu.CompilerParams(needs_layout_passes=False),
    scratch_types=[
        pltpu.VMEM((EPT,), i32),
        pltpu.VMEM((HALF,), f32),
        pltpu.VMEM_SHARED((NTILES * HALF,), f32),
        pltpu.VMEM((NTILES * RT,), f32),
        pltpu.VMEM((RT,), f32),
        pltpu.VMEM((64, D), f32),
        pltpu.VMEM((64, D), f32),
    ],
)(_k2_body)


def _layer_body(want_g, g_hbm, h_hbm, ds_hbm, src_hbm, dst_hbm,
                hn_hbm, gn_hbm, src_s, dst_s, pkc, srci, dsti, rowsA, rowsB,
                prop_sh, prop_t, h_t, hn_t, gn_t, ds_t, semA, semB,
                semSA, semSB):
    cid = lax.axis_index("c")
    sid = lax.axis_index("s")
    lo = cid * HALF
    hi = lo + HALF
    zero16 = jnp.zeros((16,), f32)
    zero16i = jnp.zeros((16,), i32)
    trash16 = jnp.full((16,), TRASH, i32)

    def z0(r, c):
        for v in range(D // 16):
            prop_t[r, pl.ds(v * 16, 16)] = zero16
        return c
    lax.fori_loop(0, EC, z0, 0)
    for cb in range(RT // EC):
        pltpu.sync_copy(prop_t, prop_sh.at[pl.ds(sid * RT + cb * EC, EC)])
    plsc.subcore_barrier()

    trashp16 = jnp.full((16,), TRASH - HALF + HALF, i32)
    def pf(r, c):
        for v in range(GC // 16):
            pkc[r, pl.ds(v * 16, 16)] = trashp16
        return c
    lax.fori_loop(0, CAP, pf, 0)

    ebase = sid * EPT
    nv = jnp.zeros((16,), i32)
    for sc in range(NSC):
        pltpu.sync_copy(src_hbm.at[pl.ds(ebase + sc * SCH, SCH)], src_s)
        pltpu.sync_copy(dst_hbm.at[pl.ds(ebase + sc * SCH, SCH)], dst_s)

        def it(i, nv):
            s16 = src_s[pl.ds(i * 16, 16)]
            d16 = dst_s[pl.ds(i * 16, 16)]
            m = (d16 >= lo) & (d16 < hi)
            cs = plsc.cumsum(m.astype(i32))
            pos = nv + cs - 1
            r = lax.shift_right_logical(pos, 7)
            col = pos & (GC - 1)
            pk = lax.shift_left(s16, SHIFT) | (d16 - lo)
            plsc.store_scatter(pkc, [r, col], pk, mask=m)
            return nv + plsc.all_reduce_population_count(m)
        nv = lax.fori_loop(0, SCH // 16, it, nv)
    n = jnp.max(nv)

    n_pr = (n + (2 * GC - 1)) // (2 * GC)

    def unpack(j, q):
        for v in range(GC // 16):
            pk = pkc[j, pl.ds(v * 16, 16)]
            srci[pl.ds(q * GC + v * 16, 16)] = lax.shift_right_logical(pk, SHIFT)
            dsti[q, pl.ds(v * 16, 16)] = pk & LOWM

    def pair(p, c):
        unpack(2 * p, 0)
        cpA = pltpu.async_copy(g_hbm.at[srci.at[pl.ds(0, GC)]], rowsA, semA)
        unpack(2 * p + 1, 1)
        cpB = pltpu.async_copy(g_hbm.at[srci.at[pl.ds(GC, GC)]], rowsB, semB)
        cpA.wait()
        pltpu.sync_copy(rowsA, prop_sh.at[dsti.at[0]], add=True)
        cpB.wait()
        pltpu.sync_copy(rowsB, prop_sh.at[dsti.at[1]], add=True)
        return c
    lax.fori_loop(0, n_pr, pair, 0)
    plsc.subcore_barrier()

    gbase = lo + sid * RT
    pltpu.sync_copy(ds_hbm.at[pl.ds(gbase, RT)], ds_t)
    for cb in range(RT // EC):
        lb = sid * RT + cb * EC
        pltpu.sync_copy(prop_sh.at[pl.ds(lb, EC)], prop_t)
        pltpu.sync_copy(h_hbm.at[pl.ds(gbase + cb * EC, EC)], h_t)

        def rb(r, c):
            dsr = plsc.load_gather(ds_t, [jnp.full((16,), cb * EC + r, i32)])
            for v in range(D // 16):
                p = prop_t[r, pl.ds(v * 16, 16)] * dsr
                t = AGGR * p + (1.0 - AGGR) * h_t[r, pl.ds(v * 16, 16)]
                o = jnp.maximum(t, ALPHA * t)
                hn_t[r, pl.ds(v * 16, 16)] = o
                if want_g:
                    gn_t[r, pl.ds(v * 16, 16)] = o * dsr
            return c
        lax.fori_loop(0, EC, rb, 0)
        pltpu.sync_copy(hn_t, hn_hbm.at[pl.ds(gbase + cb * EC, EC)])
        if want_g:
            pltpu.sync_copy(gn_t, gn_hbm.at[pl.ds(gbase + cb * EC, EC)])


def _make_layer(want_g):
    if want_g:
        out_type = [
            jax.ShapeDtypeStruct((NP, D), f32),
            jax.ShapeDtypeStruct((NP, D), f32),
        ]
        body = functools.partial(_layer_body, True)
    else:
        out_type = [jax.ShapeDtypeStruct((NP, D), f32)]

        def body(g_hbm, h_hbm, ds_hbm, src_hbm, dst_hbm, hn_hbm, *scratch):
            _layer_body(False, g_hbm, h_hbm, ds_hbm, src_hbm, dst_hbm,
                        hn_hbm, None, *scratch)
    return pl.kernel(
        body,
        out_type=out_type,
        mesh=_mesh,
        compiler_params=pltpu.CompilerParams(needs_layout_passes=False),
        scratch_types=[
            pltpu.VMEM((SCH,), i32),
            pltpu.VMEM((SCH,), i32),
            pltpu.VMEM((CAP, GC), i32),
            pltpu.VMEM((2 * GC,), i32),
            pltpu.VMEM((8, GC), i32),
            pltpu.VMEM((GC, D), f32),
            pltpu.VMEM((GC, D), f32),
            pltpu.VMEM_SHARED((HALF + 16, D), f32),
            pltpu.VMEM((EC, D), f32),
            pltpu.VMEM((EC, D), f32),
            pltpu.VMEM((EC, D), f32),
            pltpu.VMEM((EC, D), f32),
            pltpu.VMEM((RT,), f32),
            pltpu.SemaphoreType.DMA,
            pltpu.SemaphoreType.DMA,
            pltpu.SemaphoreType.DMA,
            pltpu.SemaphoreType.DMA,
        ],
    )


_k3 = _make_layer(True)
_k4 = _make_layer(False)


@jax.jit
def kernel(x, edge_index, W_lin, b_lin):
    src = edge_index[0]
    dst = edge_index[1]
    xp = jnp.pad(x, ((0, NP - N), (0, 0)))
    Wt = W_lin.T
    b8 = jnp.broadcast_to(b_lin[None, :], (8, D))
    h0 = _mm(xp, Wt, b8)
    ds, g0 = _k2(h0, dst)
    h1, g1 = _k3(g0, h0, ds, src, dst)
    (h2,) = _k4(g1, h1, ds, src, dst)
    return h2[:N]


# speedup vs baseline: 1.1959x; 1.1959x over previous
import functools

import jax
import jax.numpy as jnp
from jax import lax
from jax.experimental import pallas as pl
from jax.experimental.pallas import tpu as pltpu
from jax.experimental.pallas import tpu_sc as plsc

N = 10000
E = 320000
D = 128
NP = 10240
HALF = NP // 2
NTILES = 16
RT = HALF // NTILES
EPT = E // NTILES
SCH = 4000
NSC = EPT // SCH
GC = 128
CAP = 160
EC = 32
TRASH = HALF
SHIFT = 13
LOWM = (1 << SHIFT) - 1
ALPHA = 0.2
AGGR = 0.5

f32 = jnp.float32
i32 = jnp.int32

_mesh = plsc.VectorSubcoreMesh(core_axis_name="c", subcore_axis_name="s")


def _mm_body(x_ref, wt_ref, b_ref, o_ref):
    o_ref[...] = (
        jnp.dot(x_ref[...], wt_ref[...], preferred_element_type=f32)
        + b_ref[0:1, :]
    )


def _mm(xp, Wt, b8):
    BLK = 512
    return pl.pallas_call(
        _mm_body,
        grid=(NP // BLK,),
        in_specs=[
            pl.BlockSpec((BLK, D), lambda i: (i, 0)),
            pl.BlockSpec((D, D), lambda i: (0, 0)),
            pl.BlockSpec((8, D), lambda i: (0, 0)),
        ],
        out_specs=pl.BlockSpec((BLK, D), lambda i: (i, 0)),
        out_shape=jax.ShapeDtypeStruct((NP, D), f32),
    )(xp, Wt, b8)


def _k2_body(h0_hbm, dst_hbm, ds_hbm, g0_hbm,
             dst_s, degl, deg_sh, cmb, ds_t, h_t, g_t):
    cid = lax.axis_index("c")
    sid = lax.axis_index("s")
    lo = cid * HALF
    hi = lo + HALF
    zero16 = jnp.zeros((16,), f32)
    ones16 = jnp.ones((16,), f32)

    def zb(i, c):
        degl[pl.ds(i * 16, 16)] = zero16
        return c
    lax.fori_loop(0, HALF // 16, zb, 0)

    ebase = sid * EPT
    pltpu.sync_copy(dst_hbm.at[pl.ds(ebase, EPT)], dst_s)

    def it(i, c):
        d16 = dst_s[pl.ds(i * 16, 16)]
        m = (d16 >= lo) & (d16 < hi)
        plsc.addupdate_scatter(degl, [d16 - lo], ones16, mask=m)
        return c
    lax.fori_loop(0, EPT // 16, it, 0)

    pltpu.sync_copy(degl, deg_sh.at[pl.ds(sid * HALF, HALF)])
    plsc.subcore_barrier()
    for t in range(NTILES):
        pltpu.sync_copy(deg_sh.at[pl.ds(t * HALF + sid * RT, RT)],
                        cmb.at[pl.ds(t * RT, RT)])
    for k in range(RT // 16):
        acc = cmb[pl.ds(k * 16, 16)]
        for t in range(1, NTILES):
            acc = acc + cmb[pl.ds(t * RT + k * 16, 16)]
        dg = jnp.maximum(acc, 1.0)
        xi = plsc.bitcast(dg, i32)
        yi = 0x5F3759DF - lax.shift_right_arithmetic(xi, 1)
        y = plsc.bitcast(yi, f32)
        for _ in range(3):
            y = y * (1.5 - 0.5 * dg * y * y)
        ds_t[pl.ds(k * 16, 16)] = y
    gbase = lo + sid * RT
    pltpu.sync_copy(ds_t, ds_hbm.at[pl.ds(gbase, RT)])

    for cb in range(RT // 64):
        pltpu.sync_copy(h0_hbm.at[pl.ds(gbase + cb * 64, 64)], h_t)

        def rb(r, c):
            dsr = plsc.load_gather(ds_t, [jnp.full((16,), cb * 64 + r, i32)])
            for v in range(D // 16):
                g_t[r, pl.ds(v * 16, 16)] = h_t[r, pl.ds(v * 16, 16)] * dsr
            return c
        lax.fori_loop(0, 64, rb, 0)
        pltpu.sync_copy(g_t, g0_hbm.at[pl.ds(gbase + cb * 64, 64)])


_k2 = functools.partial(
    pl.kernel,
    out_type=[
        jax.ShapeDtypeStruct((NP,), f32),
        jax.ShapeDtypeStruct((NP, D), f32),
    ],
    mesh=_mesh,
    compiler_params=pltpu.CompilerParams(needs_layout_passes=False),
    scratch_types=[
        pltpu.VMEM((EPT,), i32),
        pltpu.VMEM((HALF,), f32),
        pltpu.VMEM_SHARED((NTILES * HALF,), f32),
        pltpu.VMEM((NTILES * RT,), f32),
        pltpu.VMEM((RT,), f32),
        pltpu.VMEM((64, D), f32),
        pltpu.VMEM((64, D), f32),
    ],
)(_k2_body)


def _layer_body(want_g, g_hbm, h_hbm, ds_hbm, src_hbm, dst_hbm,
                hn_hbm, gn_hbm, src_s, dst_s, srcc, dstc, rowsA,
                prop_sh, prop_t, h_t, h2_t, hn_t, gn_t, ds_t, semA):
    cid = lax.axis_index("c")
    sid = lax.axis_index("s")
    lo = cid * HALF
    hi = lo + HALF
    zero16 = jnp.zeros((16,), f32)
    zero16i = jnp.zeros((16,), i32)
    trash16 = jnp.full((16,), TRASH, i32)

    def z0(r, c):
        for v in range(D // 16):
            prop_t[r, pl.ds(v * 16, 16)] = zero16
        return c
    lax.fori_loop(0, EC, z0, 0)
    for cb in range(RT // EC):
        pltpu.sync_copy(prop_t, prop_sh.at[pl.ds(sid * RT + cb * EC, EC)])
    plsc.subcore_barrier()

    def pf(r, c):
        for v in range(GC // 16):
            srcc[pl.ds(r * GC + v * 16, 16)] = zero16i
            dstc[r, pl.ds(v * 16, 16)] = trash16
        return c
    lax.fori_loop(0, CAP, pf, 0)

    ebase = sid * EPT
    nv = jnp.zeros((16,), i32)
    for sc in range(NSC):
        pltpu.sync_copy(src_hbm.at[pl.ds(ebase + sc * SCH, SCH)], src_s)
        pltpu.sync_copy(dst_hbm.at[pl.ds(ebase + sc * SCH, SCH)], dst_s)

        def it(i, nv):
            s16 = src_s[pl.ds(i * 16, 16)]
            d16 = dst_s[pl.ds(i * 16, 16)]
            m = (d16 >= lo) & (d16 < hi)
            cs = plsc.cumsum(m.astype(i32))
            pos = nv + cs - 1
            r = lax.shift_right_logical(pos, 7)
            col = pos & (GC - 1)
            plsc.store_scatter(srcc, [pos], s16, mask=m)
            plsc.store_scatter(dstc, [r, col], d16 - lo, mask=m)
            return nv + plsc.all_reduce_population_count(m)
        nv = lax.fori_loop(0, SCH // 16, it, nv)
    n = jnp.max(nv)

    n_ch = (n + (GC - 1)) // GC

    def gb(j, c):
        pltpu.async_copy(g_hbm.at[srcc.at[pl.ds(j * GC, GC)]], rowsA,
                         semA).wait()
        pltpu.sync_copy(rowsA, prop_sh.at[dstc.at[j]], add=True)
        return c
    lax.fori_loop(0, n_ch, gb, 0)
    plsc.subcore_barrier()

    gbase = lo + sid * RT
    pltpu.sync_copy(ds_hbm.at[pl.ds(gbase, RT)], ds_t)
    nch = RT // EC
    hbufs = [h_t, gn_t] if not want_g else [h_t, h2_t]
    cps = [None] * nch
    cps[0] = pltpu.async_copy(h_hbm.at[pl.ds(gbase, EC)], hbufs[0], semA)
    for cb in range(nch):
        lb = sid * RT + cb * EC
        pltpu.sync_copy(prop_sh.at[pl.ds(lb, EC)], prop_t)
        if cb + 1 < nch:
            cps[cb + 1] = pltpu.async_copy(
                h_hbm.at[pl.ds(gbase + (cb + 1) * EC, EC)],
                hbufs[(cb + 1) % 2], semA)
        cps[cb].wait()
        hcur = hbufs[cb % 2]

        def rb(r, c, cb=cb, hcur=hcur):
            dsr = plsc.load_gather(ds_t, [jnp.full((16,), cb * EC + r, i32)])
            for v in range(D // 16):
                p = prop_t[r, pl.ds(v * 16, 16)] * dsr
                t = AGGR * p + (1.0 - AGGR) * hcur[r, pl.ds(v * 16, 16)]
                o = jnp.maximum(t, ALPHA * t)
                hn_t[r, pl.ds(v * 16, 16)] = o
                if want_g:
                    gn_t[r, pl.ds(v * 16, 16)] = o * dsr
            return c
        lax.fori_loop(0, EC, rb, 0)
        pltpu.sync_copy(hn_t, hn_hbm.at[pl.ds(gbase + cb * EC, EC)])
        if want_g:
            pltpu.sync_copy(gn_t, gn_hbm.at[pl.ds(gbase + cb * EC, EC)])


def _make_layer(want_g):
    if want_g:
        out_type = [
            jax.ShapeDtypeStruct((NP, D), f32),
            jax.ShapeDtypeStruct((NP, D), f32),
        ]
        body = functools.partial(_layer_body, True)
    else:
        out_type = [jax.ShapeDtypeStruct((NP, D), f32)]

        def body(g_hbm, h_hbm, ds_hbm, src_hbm, dst_hbm, hn_hbm, *scratch):
            _layer_body(False, g_hbm, h_hbm, ds_hbm, src_hbm, dst_hbm,
                        hn_hbm, None, *scratch)
    return pl.kernel(
        body,
        out_type=out_type,
        mesh=_mesh,
        compiler_params=pltpu.CompilerParams(needs_layout_passes=False),
        scratch_types=[
            pltpu.VMEM((SCH,), i32),
            pltpu.VMEM((SCH,), i32),
            pltpu.VMEM((CAP * GC,), i32),
            pltpu.VMEM((CAP, GC), i32),
            pltpu.VMEM((GC, D), f32),
            pltpu.VMEM_SHARED((HALF + 16, D), f32),
            pltpu.VMEM((EC, D), f32),
            pltpu.VMEM((EC, D), f32),
            pltpu.VMEM((EC, D), f32),
            pltpu.VMEM((EC, D), f32),
            pltpu.VMEM((EC, D), f32),
            pltpu.VMEM((RT,), f32),
            pltpu.SemaphoreType.DMA,
        ],
    )


_k3 = _make_layer(True)
_k4 = _make_layer(False)


@jax.jit
def kernel(x, edge_index, W_lin, b_lin):
    src = edge_index[0]
    dst = edge_index[1]
    xp = jnp.pad(x, ((0, NP - N), (0, 0)))
    Wt = W_lin.T
    b8 = jnp.broadcast_to(b_lin[None, :], (8, D))
    h0 = _mm(xp, Wt, b8)
    ds, g0 = _k2(h0, dst)
    h1, g1 = _k3(g0, h0, ds, src, dst)
    (h2,) = _k4(g1, h1, ds, src, dst)
    return h2[:N]


# speedup vs baseline: 1.2068x; 1.0091x over previous
import functools

import jax
import jax.numpy as jnp
from jax import lax
from jax.experimental import pallas as pl
from jax.experimental.pallas import tpu as pltpu
from jax.experimental.pallas import tpu_sc as plsc

N = 10000
E = 320000
D = 128
NP = 10240
HALF = NP // 2
NTILES = 16
RT = HALF // NTILES
EPT = E // NTILES
SCH = 4000
NSC = EPT // SCH
GC = 128
CAP = 160
EC = 32
TRASH = HALF
SHIFT = 13
LOWM = (1 << SHIFT) - 1
ALPHA = 0.2
AGGR = 0.5

f32 = jnp.float32
i32 = jnp.int32

_mesh = plsc.VectorSubcoreMesh(core_axis_name="c", subcore_axis_name="s")


def _mm_body(x_ref, wt_ref, b_ref, o_ref):
    o_ref[...] = (
        jnp.dot(x_ref[...], wt_ref[...], preferred_element_type=f32)
        + b_ref[0:1, :]
    )


def _mm(xp, Wt, b8):
    BLK = 512
    return pl.pallas_call(
        _mm_body,
        grid=(NP // BLK,),
        in_specs=[
            pl.BlockSpec((BLK, D), lambda i: (i, 0)),
            pl.BlockSpec((D, D), lambda i: (0, 0)),
            pl.BlockSpec((8, D), lambda i: (0, 0)),
        ],
        out_specs=pl.BlockSpec((BLK, D), lambda i: (i, 0)),
        out_shape=jax.ShapeDtypeStruct((NP, D), f32),
    )(xp, Wt, b8)


def _k2_body(h0_hbm, dst_hbm, ds_hbm, g0_hbm,
             dst_s, degl, deg_sh, cmb, ds_t, h_t, g_t):
    cid = lax.axis_index("c")
    sid = lax.axis_index("s")
    lo = cid * HALF
    hi = lo + HALF
    zero16 = jnp.zeros((16,), f32)
    ones16 = jnp.ones((16,), f32)

    def zb(i, c):
        degl[pl.ds(i * 16, 16)] = zero16
        return c
    lax.fori_loop(0, HALF // 16, zb, 0)

    ebase = sid * EPT
    pltpu.sync_copy(dst_hbm.at[pl.ds(ebase, EPT)], dst_s)

    def it(i, c):
        d16 = dst_s[pl.ds(i * 16, 16)]
        m = (d16 >= lo) & (d16 < hi)
        plsc.addupdate_scatter(degl, [d16 - lo], ones16, mask=m)
        return c
    lax.fori_loop(0, EPT // 16, it, 0)

    pltpu.sync_copy(degl, deg_sh.at[pl.ds(sid * HALF, HALF)])
    plsc.subcore_barrier()
    for t in range(NTILES):
        pltpu.sync_copy(deg_sh.at[pl.ds(t * HALF + sid * RT, RT)],
                        cmb.at[pl.ds(t * RT, RT)])
    for k in range(RT // 16):
        acc = cmb[pl.ds(k * 16, 16)]
        for t in range(1, NTILES):
            acc = acc + cmb[pl.ds(t * RT + k * 16, 16)]
        dg = jnp.maximum(acc, 1.0)
        xi = plsc.bitcast(dg, i32)
        yi = 0x5F3759DF - lax.shift_right_arithmetic(xi, 1)
        y = plsc.bitcast(yi, f32)
        for _ in range(3):
            y = y * (1.5 - 0.5 * dg * y * y)
        ds_t[pl.ds(k * 16, 16)] = y
    gbase = lo + sid * RT
    pltpu.sync_copy(ds_t, ds_hbm.at[pl.ds(gbase, RT)])

    for cb in range(RT // 64):
        pltpu.sync_copy(h0_hbm.at[pl.ds(gbase + cb * 64, 64)], h_t)

        def rb(r, c):
            dsr = plsc.load_gather(ds_t, [jnp.full((16,), cb * 64 + r, i32)])
            for v in range(D // 16):
                g_t[r, pl.ds(v * 16, 16)] = h_t[r, pl.ds(v * 16, 16)] * dsr
            return c
        lax.fori_loop(0, 64, rb, 0)
        pltpu.sync_copy(g_t, g0_hbm.at[pl.ds(gbase + cb * 64, 64)])


_k2 = functools.partial(
    pl.kernel,
    out_type=[
        jax.ShapeDtypeStruct((NP,), f32),
        jax.ShapeDtypeStruct((NP, D), f32),
    ],
    mesh=_mesh,
    compiler_params=pltpu.CompilerParams(needs_layout_passes=False),
    scratch_types=[
        pltpu.VMEM((EPT,), i32),
        pltpu.VMEM((HALF,), f32),
        pltpu.VMEM_SHARED((NTILES * HALF,), f32),
        pltpu.VMEM((NTILES * RT,), f32),
        pltpu.VMEM((RT,), f32),
        pltpu.VMEM((64, D), f32),
        pltpu.VMEM((64, D), f32),
    ],
)(_k2_body)


def _layer_body(want_g, g_hbm, h_hbm, ds_hbm, src_hbm, dst_hbm,
                hn_hbm, gn_hbm, src_s, dst_s, srcc, dstc, rowsA,
                prop_sh, prop_t, h_t, h2_t, hn_t, gn_t, ds_t, semA):
    cid = lax.axis_index("c")
    sid = lax.axis_index("s")
    lo = cid * HALF
    hi = lo + HALF
    zero16 = jnp.zeros((16,), f32)
    zero16i = jnp.zeros((16,), i32)
    trash16 = jnp.full((16,), TRASH, i32)

    def z0(r, c):
        for v in range(D // 16):
            prop_t[r, pl.ds(v * 16, 16)] = zero16
        return c
    lax.fori_loop(0, EC, z0, 0)
    for cb in range(RT // EC):
        pltpu.sync_copy(prop_t, prop_sh.at[pl.ds(sid * RT + cb * EC, EC)])
    plsc.subcore_barrier()

    def pf(r, c):
        for v in range(GC // 16):
            srcc[pl.ds(r * GC + v * 16, 16)] = zero16i
            dstc[r, pl.ds(v * 16, 16)] = trash16
        return c
    lax.fori_loop(0, CAP, pf, 0)

    ebase = sid * EPT
    nv = jnp.zeros((16,), i32)
    for sc in range(NSC):
        cp1 = pltpu.async_copy(src_hbm.at[pl.ds(ebase + sc * SCH, SCH)],
                               src_s, semA)
        cp2 = pltpu.async_copy(dst_hbm.at[pl.ds(ebase + sc * SCH, SCH)],
                               dst_s, semA)
        cp1.wait()
        cp2.wait()

        def it(i, nv):
            s16 = src_s[pl.ds(i * 16, 16)]
            d16 = dst_s[pl.ds(i * 16, 16)]
            m = (d16 >= lo) & (d16 < hi)
            cs = plsc.cumsum(m.astype(i32))
            pos = nv + cs - 1
            r = lax.shift_right_logical(pos, 7)
            col = pos & (GC - 1)
            plsc.store_scatter(srcc, [pos], s16, mask=m)
            plsc.store_scatter(dstc, [r, col], d16 - lo, mask=m)
            return nv + plsc.all_reduce_population_count(m)
        nv = lax.fori_loop(0, SCH // 16, it, nv)
    n = jnp.max(nv)

    n_ch = (n + (GC - 1)) // GC

    def gb(j, c):
        pltpu.async_copy(g_hbm.at[srcc.at[pl.ds(j * GC, GC)]], rowsA,
                         semA).wait()
        pltpu.sync_copy(rowsA, prop_sh.at[dstc.at[j]], add=True)
        return c
    lax.fori_loop(0, n_ch, gb, 0)
    plsc.subcore_barrier()

    gbase = lo + sid * RT
    pltpu.sync_copy(ds_hbm.at[pl.ds(gbase, RT)], ds_t)
    nch = RT // EC
    hbufs = [h_t, gn_t] if not want_g else [h_t, h2_t]
    cps = [None] * nch
    cps[0] = pltpu.async_copy(h_hbm.at[pl.ds(gbase, EC)], hbufs[0], semA)
    for cb in range(nch):
        lb = sid * RT + cb * EC
        pltpu.sync_copy(prop_sh.at[pl.ds(lb, EC)], prop_t)
        if cb + 1 < nch:
            cps[cb + 1] = pltpu.async_copy(
                h_hbm.at[pl.ds(gbase + (cb + 1) * EC, EC)],
                hbufs[(cb + 1) % 2], semA)
        cps[cb].wait()
        hcur = hbufs[cb % 2]

        def rb(r, c, cb=cb, hcur=hcur):
            dsr = plsc.load_gather(ds_t, [jnp.full((16,), cb * EC + r, i32)])
            for v in range(D // 16):
                p = prop_t[r, pl.ds(v * 16, 16)] * dsr
                t = AGGR * p + (1.0 - AGGR) * hcur[r, pl.ds(v * 16, 16)]
                o = jnp.maximum(t, ALPHA * t)
                hn_t[r, pl.ds(v * 16, 16)] = o
                if want_g:
                    gn_t[r, pl.ds(v * 16, 16)] = o * dsr
            return c
        lax.fori_loop(0, EC, rb, 0)
        pltpu.sync_copy(hn_t, hn_hbm.at[pl.ds(gbase + cb * EC, EC)])
        if want_g:
            pltpu.sync_copy(gn_t, gn_hbm.at[pl.ds(gbase + cb * EC, EC)])


def _make_layer(want_g):
    if want_g:
        out_type = [
            jax.ShapeDtypeStruct((NP, D), f32),
            jax.ShapeDtypeStruct((NP, D), f32),
        ]
        body = functools.partial(_layer_body, True)
    else:
        out_type = [jax.ShapeDtypeStruct((NP, D), f32)]

        def body(g_hbm, h_hbm, ds_hbm, src_hbm, dst_hbm, hn_hbm, *scratch):
            _layer_body(False, g_hbm, h_hbm, ds_hbm, src_hbm, dst_hbm,
                        hn_hbm, None, *scratch)
    return pl.kernel(
        body,
        out_type=out_type,
        mesh=_mesh,
        compiler_params=pltpu.CompilerParams(needs_layout_passes=False),
        scratch_types=[
            pltpu.VMEM((SCH,), i32),
            pltpu.VMEM((SCH,), i32),
            pltpu.VMEM((CAP * GC,), i32),
            pltpu.VMEM((CAP, GC), i32),
            pltpu.VMEM((GC, D), f32),
            pltpu.VMEM_SHARED((HALF + 16, D), f32),
            pltpu.VMEM((EC, D), f32),
            pltpu.VMEM((EC, D), f32),
            pltpu.VMEM((EC, D), f32),
            pltpu.VMEM((EC, D), f32),
            pltpu.VMEM((EC, D), f32),
            pltpu.VMEM((RT,), f32),
            pltpu.SemaphoreType.DMA,
        ],
    )


_k3 = _make_layer(True)
_k4 = _make_layer(False)


@jax.jit
def kernel(x, edge_index, W_lin, b_lin):
    src = edge_index[0]
    dst = edge_index[1]
    xp = jnp.pad(x, ((0, NP - N), (0, 0)))
    Wt = W_lin.T
    b8 = jnp.broadcast_to(b_lin[None, :], (8, D))
    h0 = _mm(xp, Wt, b8)
    ds, g0 = _k2(h0, dst)
    h1, g1 = _k3(g0, h0, ds, src, dst)
    (h2,) = _k4(g1, h1, ds, src, dst)
    return h2[:N]


# speedup vs baseline: 1.2150x; 1.0068x over previous
import functools

import jax
import jax.numpy as jnp
from jax import lax
from jax.experimental import pallas as pl
from jax.experimental.pallas import tpu as pltpu
from jax.experimental.pallas import tpu_sc as plsc

N = 10000
E = 320000
D = 128
NP = 10240
HALF = NP // 2
NTILES = 16
RT = HALF // NTILES
EPT = E // NTILES
SCH = 4000
NSC = EPT // SCH
GC = 128
CAP = 160
EC = 32
TRASH = HALF
SHIFT = 13
LOWM = (1 << SHIFT) - 1
ALPHA = 0.2
AGGR = 0.5

f32 = jnp.float32
i32 = jnp.int32

_mesh = plsc.VectorSubcoreMesh(core_axis_name="c", subcore_axis_name="s")


def _mm_body(x_ref, wt_ref, b_ref, o_ref):
    o_ref[...] = (
        jnp.dot(x_ref[...], wt_ref[...], preferred_element_type=f32)
        + b_ref[0:1, :]
    )


def _mm(xp, Wt, b8):
    BLK = 512
    return pl.pallas_call(
        _mm_body,
        grid=(NP // BLK,),
        in_specs=[
            pl.BlockSpec((BLK, D), lambda i: (i, 0)),
            pl.BlockSpec((D, D), lambda i: (0, 0)),
            pl.BlockSpec((8, D), lambda i: (0, 0)),
        ],
        out_specs=pl.BlockSpec((BLK, D), lambda i: (i, 0)),
        out_shape=jax.ShapeDtypeStruct((NP, D), f32),
    )(xp, Wt, b8)


def _k2_body(h0_hbm, dst_hbm, ds_hbm, g0_hbm,
             dst_s, degl, deg_sh, cmb, ds_t, h_t, h2_t, g_t, sem):
    cid = lax.axis_index("c")
    sid = lax.axis_index("s")
    lo = cid * HALF
    hi = lo + HALF
    zero16 = jnp.zeros((16,), f32)
    ones16 = jnp.ones((16,), f32)

    def zb(i, c):
        degl[pl.ds(i * 16, 16)] = zero16
        return c
    lax.fori_loop(0, HALF // 16, zb, 0)

    ebase = sid * EPT
    pltpu.sync_copy(dst_hbm.at[pl.ds(ebase, EPT)], dst_s)

    def it(i, c):
        d16 = dst_s[pl.ds(i * 16, 16)]
        m = (d16 >= lo) & (d16 < hi)
        plsc.addupdate_scatter(degl, [d16 - lo], ones16, mask=m)
        return c
    lax.fori_loop(0, EPT // 16, it, 0)

    pltpu.sync_copy(degl, deg_sh.at[pl.ds(sid * HALF, HALF)])
    plsc.subcore_barrier()
    for t in range(NTILES):
        pltpu.sync_copy(deg_sh.at[pl.ds(t * HALF + sid * RT, RT)],
                        cmb.at[pl.ds(t * RT, RT)])
    for k in range(RT // 16):
        acc = cmb[pl.ds(k * 16, 16)]
        for t in range(1, NTILES):
            acc = acc + cmb[pl.ds(t * RT + k * 16, 16)]
        dg = jnp.maximum(acc, 1.0)
        xi = plsc.bitcast(dg, i32)
        yi = 0x5F3759DF - lax.shift_right_arithmetic(xi, 1)
        y = plsc.bitcast(yi, f32)
        for _ in range(3):
            y = y * (1.5 - 0.5 * dg * y * y)
        ds_t[pl.ds(k * 16, 16)] = y
    gbase = lo + sid * RT
    pltpu.sync_copy(ds_t, ds_hbm.at[pl.ds(gbase, RT)])

    nch = RT // 64
    hbufs = [h_t, h2_t]
    cps = [None] * nch
    cps[0] = pltpu.async_copy(h0_hbm.at[pl.ds(gbase, 64)], hbufs[0], sem)
    for cb in range(nch):
        if cb + 1 < nch:
            cps[cb + 1] = pltpu.async_copy(
                h0_hbm.at[pl.ds(gbase + (cb + 1) * 64, 64)],
                hbufs[(cb + 1) % 2], sem)
        cps[cb].wait()
        hcur = hbufs[cb % 2]

        def rb(r, c, cb=cb, hcur=hcur):
            dsr = plsc.load_gather(ds_t, [jnp.full((16,), cb * 64 + r, i32)])
            for v in range(D // 16):
                g_t[r, pl.ds(v * 16, 16)] = hcur[r, pl.ds(v * 16, 16)] * dsr
            return c
        lax.fori_loop(0, 64, rb, 0)
        pltpu.sync_copy(g_t, g0_hbm.at[pl.ds(gbase + cb * 64, 64)])


_k2 = functools.partial(
    pl.kernel,
    out_type=[
        jax.ShapeDtypeStruct((NP,), f32),
        jax.ShapeDtypeStruct((NP, D), f32),
    ],
    mesh=_mesh,
    compiler_params=pltpu.CompilerParams(needs_layout_passes=False),
    scratch_types=[
        pltpu.VMEM((EPT,), i32),
        pltpu.VMEM((HALF,), f32),
        pltpu.VMEM_SHARED((NTILES * HALF,), f32),
        pltpu.VMEM((NTILES * RT,), f32),
        pltpu.VMEM((RT,), f32),
        pltpu.VMEM((64, D), f32),
        pltpu.VMEM((64, D), f32),
        pltpu.VMEM((64, D), f32),
        pltpu.SemaphoreType.DMA,
    ],
)(_k2_body)


def _layer_body(want_g, g_hbm, h_hbm, ds_hbm, src_hbm, dst_hbm,
                hn_hbm, gn_hbm, src_s, dst_s, srcc, dstc, rowsA,
                prop_sh, prop_t, h_t, h2_t, hn_t, gn_t, ds_t, semA):
    cid = lax.axis_index("c")
    sid = lax.axis_index("s")
    lo = cid * HALF
    hi = lo + HALF
    zero16 = jnp.zeros((16,), f32)
    zero16i = jnp.zeros((16,), i32)
    trash16 = jnp.full((16,), TRASH, i32)

    def z0(r, c):
        for v in range(D // 16):
            prop_t[r, pl.ds(v * 16, 16)] = zero16
        return c
    lax.fori_loop(0, EC, z0, 0)
    for cb in range(RT // EC):
        pltpu.sync_copy(prop_t, prop_sh.at[pl.ds(sid * RT + cb * EC, EC)])
    plsc.subcore_barrier()

    def pf(r, c):
        for v in range(GC // 16):
            srcc[pl.ds(r * GC + v * 16, 16)] = zero16i
            dstc[r, pl.ds(v * 16, 16)] = trash16
        return c
    lax.fori_loop(0, CAP, pf, 0)

    ebase = sid * EPT
    nv = jnp.zeros((16,), i32)
    for sc in range(NSC):
        cp1 = pltpu.async_copy(src_hbm.at[pl.ds(ebase + sc * SCH, SCH)],
                               src_s, semA)
        cp2 = pltpu.async_copy(dst_hbm.at[pl.ds(ebase + sc * SCH, SCH)],
                               dst_s, semA)
        cp1.wait()
        cp2.wait()

        def it(i, nv):
            s16 = src_s[pl.ds(i * 16, 16)]
            d16 = dst_s[pl.ds(i * 16, 16)]
            m = (d16 >= lo) & (d16 < hi)
            cs = plsc.cumsum(m.astype(i32))
            pos = nv + cs - 1
            r = lax.shift_right_logical(pos, 7)
            col = pos & (GC - 1)
            plsc.store_scatter(srcc, [pos], s16, mask=m)
            plsc.store_scatter(dstc, [r, col], d16 - lo, mask=m)
            return nv + plsc.all_reduce_population_count(m)
        nv = lax.fori_loop(0, SCH // 16, it, nv)
    n = jnp.max(nv)

    n_ch = (n + (GC - 1)) // GC

    def gb(j, c):
        pltpu.async_copy(g_hbm.at[srcc.at[pl.ds(j * GC, GC)]], rowsA,
                         semA).wait()
        pltpu.sync_copy(rowsA, prop_sh.at[dstc.at[j]], add=True)
        return c
    lax.fori_loop(0, n_ch, gb, 0)
    plsc.subcore_barrier()

    gbase = lo + sid * RT
    pltpu.sync_copy(ds_hbm.at[pl.ds(gbase, RT)], ds_t)
    nch = RT // EC
    hbufs = [h_t, gn_t] if not want_g else [h_t, h2_t]
    cps = [None] * nch
    cps[0] = pltpu.async_copy(h_hbm.at[pl.ds(gbase, EC)], hbufs[0], semA)
    for cb in range(nch):
        lb = sid * RT + cb * EC
        pltpu.sync_copy(prop_sh.at[pl.ds(lb, EC)], prop_t)
        if cb + 1 < nch:
            cps[cb + 1] = pltpu.async_copy(
                h_hbm.at[pl.ds(gbase + (cb + 1) * EC, EC)],
                hbufs[(cb + 1) % 2], semA)
        cps[cb].wait()
        hcur = hbufs[cb % 2]

        def rb(r, c, cb=cb, hcur=hcur):
            dsr = plsc.load_gather(ds_t, [jnp.full((16,), cb * EC + r, i32)])
            for v in range(D // 16):
                p = prop_t[r, pl.ds(v * 16, 16)] * dsr
                t = AGGR * p + (1.0 - AGGR) * hcur[r, pl.ds(v * 16, 16)]
                o = jnp.maximum(t, ALPHA * t)
                hn_t[r, pl.ds(v * 16, 16)] = o
                if want_g:
                    gn_t[r, pl.ds(v * 16, 16)] = o * dsr
            return c
        lax.fori_loop(0, EC, rb, 0)
        pltpu.sync_copy(hn_t, hn_hbm.at[pl.ds(gbase + cb * EC, EC)])
        if want_g:
            pltpu.sync_copy(gn_t, gn_hbm.at[pl.ds(gbase + cb * EC, EC)])


def _make_layer(want_g):
    if want_g:
        out_type = [
            jax.ShapeDtypeStruct((NP, D), f32),
            jax.ShapeDtypeStruct((NP, D), f32),
        ]
        body = functools.partial(_layer_body, True)
    else:
        out_type = [jax.ShapeDtypeStruct((NP, D), f32)]

        def body(g_hbm, h_hbm, ds_hbm, src_hbm, dst_hbm, hn_hbm, *scratch):
            _layer_body(False, g_hbm, h_hbm, ds_hbm, src_hbm, dst_hbm,
                        hn_hbm, None, *scratch)
    return pl.kernel(
        body,
        out_type=out_type,
        mesh=_mesh,
        compiler_params=pltpu.CompilerParams(needs_layout_passes=False),
        scratch_types=[
            pltpu.VMEM((SCH,), i32),
            pltpu.VMEM((SCH,), i32),
            pltpu.VMEM((CAP * GC,), i32),
            pltpu.VMEM((CAP, GC), i32),
            pltpu.VMEM((GC, D), f32),
            pltpu.VMEM_SHARED((HALF + 16, D), f32),
            pltpu.VMEM((EC, D), f32),
            pltpu.VMEM((EC, D), f32),
            pltpu.VMEM((EC, D), f32),
            pltpu.VMEM((EC, D), f32),
            pltpu.VMEM((EC, D), f32),
            pltpu.VMEM((RT,), f32),
            pltpu.SemaphoreType.DMA,
        ],
    )


_k3 = _make_layer(True)
_k4 = _make_layer(False)


@jax.jit
def kernel(x, edge_index, W_lin, b_lin):
    src = edge_index[0]
    dst = edge_index[1]
    xp = jnp.pad(x, ((0, NP - N), (0, 0)))
    Wt = W_lin.T
    b8 = jnp.broadcast_to(b_lin[None, :], (8, D))
    h0 = _mm(xp, Wt, b8)
    ds, g0 = _k2(h0, dst)
    h1, g1 = _k3(g0, h0, ds, src, dst)
    (h2,) = _k4(g1, h1, ds, src, dst)
    return h2[:N]
